# Initial kernel scaffold; baseline (speedup 1.0000x reference)
import jax
import jax.numpy as jnp
from jax.experimental import pallas as pl


def kernel(x, edge_index, W1, a_src1, a_dst1, b1, W2, a_src2, a_dst2, b2):
    raise NotImplementedError("write your pallas kernel here")


# speedup vs baseline: 14.6952x; 14.6952x over previous
import functools

import jax
import jax.numpy as jnp
from jax import lax
from jax.experimental import pallas as pl
from jax.experimental.pallas import tpu as pltpu
from jax.experimental.pallas import tpu_sc as plsc

N = 10000
E = 320000
F_IN = 128
HID = 64
HEADS = 8
C = 64

NPAD = 10240
EPAD = 327680
NC, NS, L = 2, 16, 16
SL = NPAD // NS
EPT = EPAD // NS
BLK = 2048
NBLK = EPT // BLK
CH = 128
NCH = BLK // CH
NEG_SLOPE = 0.2

f32 = jnp.float32
i32 = jnp.int32


def _prep1_body(x_ref, w_ref, asr_ref, adr_ref, ht_ref, ast_ref, adt_ref):
    h = jnp.dot(x_ref[...], w_ref[...], preferred_element_type=f32)
    for hh in range(HEADS):
        blk = h[:, hh * HID:(hh + 1) * HID]
        ht_ref[hh] = blk
        ast_ref[hh] = jnp.sum(blk * asr_ref[hh][None, :], axis=1)
        adt_ref[hh] = jnp.sum(blk * adr_ref[hh][None, :], axis=1)


def _prep1(xp, W1, a_src, a_dst):
    BR = 1024
    nb = NPAD // BR
    return pl.pallas_call(
        _prep1_body,
        grid=(nb,),
        in_specs=[
            pl.BlockSpec((BR, F_IN), lambda i: (i, 0)),
            pl.BlockSpec((F_IN, HEADS * HID), lambda i: (0, 0)),
            pl.BlockSpec((HEADS, HID), lambda i: (0, 0)),
            pl.BlockSpec((HEADS, HID), lambda i: (0, 0)),
        ],
        out_specs=[
            pl.BlockSpec((HEADS, BR, HID), lambda i: (0, i, 0)),
            pl.BlockSpec((HEADS, BR), lambda i: (0, i)),
            pl.BlockSpec((HEADS, BR), lambda i: (0, i)),
        ],
        out_shape=[
            jax.ShapeDtypeStruct((HEADS, NPAD, HID), f32),
            jax.ShapeDtypeStruct((HEADS, NPAD), f32),
            jax.ShapeDtypeStruct((HEADS, NPAD), f32),
        ],
    )(xp, W1, a_src, a_dst)


def _prep2_body(o1_ref, b1_ref, w2_ref, asr_ref, adr_ref,
                ht_ref, ast_ref, adt_ref):
    acc = jnp.zeros((o1_ref.shape[1], HEADS * C), f32)
    for hh in range(HEADS):
        g = o1_ref[hh] + b1_ref[hh][None, :]
        g = jnp.where(g > 0, g, jnp.exp(g) - 1.0)
        acc = acc + jnp.dot(g, w2_ref[hh], preferred_element_type=f32)
    for hh in range(HEADS):
        blk = acc[:, hh * C:(hh + 1) * C]
        ht_ref[hh] = blk
        ast_ref[hh] = jnp.sum(blk * asr_ref[hh][None, :], axis=1)
        adt_ref[hh] = jnp.sum(blk * adr_ref[hh][None, :], axis=1)


def _prep2(out1, b1r, W2r, a_src, a_dst):
    BR = 1024
    nb = NPAD // BR
    return pl.pallas_call(
        _prep2_body,
        grid=(nb,),
        in_specs=[
            pl.BlockSpec((HEADS, BR, HID), lambda i: (0, i, 0)),
            pl.BlockSpec((HEADS, HID), lambda i: (0, 0)),
            pl.BlockSpec((HEADS, HID, HEADS * C), lambda i: (0, 0, 0)),
            pl.BlockSpec((HEADS, C), lambda i: (0, 0)),
            pl.BlockSpec((HEADS, C), lambda i: (0, 0)),
        ],
        out_specs=[
            pl.BlockSpec((HEADS, BR, C), lambda i: (0, i, 0)),
            pl.BlockSpec((HEADS, BR), lambda i: (0, i)),
            pl.BlockSpec((HEADS, BR), lambda i: (0, i)),
        ],
        out_shape=[
            jax.ShapeDtypeStruct((HEADS, NPAD, C), f32),
            jax.ShapeDtypeStruct((HEADS, NPAD), f32),
            jax.ShapeDtypeStruct((HEADS, NPAD), f32),
        ],
    )(out1, b1r, W2r, a_src, a_dst)


def _final_body(p_ref, b2_ref, o_ref):
    o_ref[...] = p_ref[0] + p_ref[1] + b2_ref[...]


def _final(part, b2r):
    BR = 1024
    nb = NPAD // BR
    return pl.pallas_call(
        _final_body,
        grid=(nb,),
        in_specs=[
            pl.BlockSpec((2, BR, C), lambda i: (0, i, 0)),
            pl.BlockSpec((1, C), lambda i: (0, 0)),
        ],
        out_specs=pl.BlockSpec((BR, C), lambda i: (i, 0)),
        out_shape=jax.ShapeDtypeStruct((NPAD, C), f32),
    )(part, b2r)


def _make_sc_layer(concat):
    hpc = HEADS // NC
    scale = 1.0 if concat else 1.0 / HEADS
    out_shape = (jax.ShapeDtypeStruct((HEADS, NPAD, HID), f32) if concat
                 else jax.ShapeDtypeStruct((NC, NPAD, C), f32))
    mesh = plsc.VectorSubcoreMesh(core_axis_name="c", subcore_axis_name="s",
                                  num_cores=NC, num_subcores=NS)

    @functools.partial(
        pl.kernel, mesh=mesh, out_type=out_shape,
        compiler_params=pltpu.CompilerParams(needs_layout_passes=False,
                                             use_tc_tiling_on_sc=False),
        scratch_types=[
            pltpu.VMEM((NCH, CH), i32),
            pltpu.VMEM((NCH, CH), i32),
            pltpu.VMEM((NCH, CH), i32),
            pltpu.VMEM((CH,), f32),
            pltpu.VMEM((NPAD,), f32),
            pltpu.VMEM((NPAD,), f32),
            pltpu.VMEM((NPAD,), f32),
            pltpu.VMEM((NS, SL), f32),
            pltpu.VMEM((SL,), f32),
            pltpu.VMEM((CH, HID), f32),
            pltpu.VMEM((CH, HID), f32),
            pltpu.VMEM_SHARED((NPAD, HID), f32),
            pltpu.VMEM_SHARED((NS, NPAD), f32),
            pltpu.VMEM_SHARED((NPAD,), f32),
            pltpu.SemaphoreType.DMA,
        ],
    )
    def sck(src_hbm, dst_hbm, ast_hbm, adt_hbm, ht_hbm, out_hbm,
            src_v, dst_v, adj_v, coef_v, as_v, ad_v, den_v, mrg_v, red_v,
            rows_v, z_v, out_sp, den_sp, mer_sp, sem):
        c = lax.axis_index("c")
        s = lax.axis_index("s")
        zero16 = jnp.zeros((L,), f32)

        def zrow(i, _):
            for q in range(HID // L):
                z_v[i, pl.ds(q * L, L)] = zero16
            return 0
        lax.fori_loop(0, CH, zrow, 0)
        for k in range(SL // CH):
            pltpu.sync_copy(z_v, out_sp.at[pl.ds(s * SL + k * CH, CH), :])

        def stage(b, _):
            row0 = s * (EPT // CH) + b * NCH
            pltpu.sync_copy(src_hbm.at[pl.ds(row0, NCH), :], src_v)
            pltpu.sync_copy(dst_hbm.at[pl.ds(row0, NCH), :], dst_v)
            return 0

        def edge_coef(j, v):
            sl = pl.ds(v * L, L)
            s16 = src_v[j, sl]
            d16 = dst_v[j, sl]
            a = plsc.load_gather(as_v, [s16]) + plsc.load_gather(ad_v, [d16])
            a = jnp.where(a > 0, a, NEG_SLOPE * a)
            return s16, d16, jnp.exp(a)

        def head_step(hh, _):
            head = c * hpc + hh
            def dz(i, _):
                den_v[pl.ds(i * L, L)] = zero16
                return 0
            lax.fori_loop(0, NPAD // L, dz, 0)
            pltpu.sync_copy(ast_hbm.at[head], as_v)
            pltpu.sync_copy(adt_hbm.at[head], ad_v)

            def p1_block(b, _):
                stage(b, 0)

                def p1_chunk(j, _):
                    for v in range(CH // L):
                        _, d16, e = edge_coef(j, v)
                        plsc.addupdate_scatter(den_v, [d16], e)
                    return 0
                lax.fori_loop(0, NCH, p1_chunk, 0)
                return 0
            lax.fori_loop(0, NBLK, p1_block, 0)

            pltpu.sync_copy(den_v, den_sp.at[s])
            plsc.subcore_barrier()
            pltpu.sync_copy(den_sp.at[:, pl.ds(s * SL, SL)], mrg_v)

            def dred(v2, _):
                sl = pl.ds(v2 * L, L)
                acc = mrg_v[0, sl]
                for r in range(1, NS):
                    acc = acc + mrg_v[r, sl]
                red_v[sl] = acc
                return 0
            lax.fori_loop(0, SL // L, dred, 0)
            pltpu.sync_copy(red_v, mer_sp.at[pl.ds(s * SL, SL)])
            plsc.subcore_barrier()
            pltpu.sync_copy(mer_sp, den_v)

            off = head * NPAD

            def p2_block(b, _):
                stage(b, 0)

                def p2_chunk(j, _):
                    for v in range(CH // L):
                        s16, d16, e = edge_coef(j, v)
                        den = plsc.load_gather(den_v, [d16])
                        cf = e / (den + 1e-16) * scale
                        coef_v[pl.ds(v * L, L)] = cf
                        adj_v[j, pl.ds(v * L, L)] = s16 + off
                    pltpu.async_copy(ht_hbm.at[adj_v.at[j]], rows_v, sem).wait()

                    def rscale(i, _):
                        bc = plsc.load_gather(
                            coef_v, [jnp.full((L,), i, dtype=i32)])
                        for q in range(HID // L):
                            sq = pl.ds(q * L, L)
                            rows_v[i, sq] = rows_v[i, sq] * bc
                        return 0
                    lax.fori_loop(0, CH, rscale, 0)
                    pltpu.sync_copy(rows_v, out_sp.at[dst_v.at[j]], add=True)
                    return 0
                lax.fori_loop(0, NCH, p2_chunk, 0)
                return 0
            lax.fori_loop(0, NBLK, p2_block, 0)

            if concat:
                plsc.subcore_barrier()
                pltpu.sync_copy(out_sp.at[pl.ds(s * SL, SL), :],
                                out_hbm.at[head, pl.ds(s * SL, SL), :])
                for k in range(SL // CH):
                    pltpu.sync_copy(
                        z_v, out_sp.at[pl.ds(s * SL + k * CH, CH), :])
            return 0

        lax.fori_loop(0, hpc, head_step, 0)

        if not concat:
            plsc.subcore_barrier()
            pltpu.sync_copy(out_sp.at[pl.ds(s * SL, SL), :],
                            out_hbm.at[c, pl.ds(s * SL, SL), :])

    return sck


_sc_layer1 = _make_sc_layer(concat=True)
_sc_layer2 = _make_sc_layer(concat=False)


def kernel(x, edge_index, W1, a_src1, a_dst1, b1, W2, a_src2, a_dst2, b2):
    xp = jnp.pad(x, ((0, NPAD - N), (0, 0)))
    ei = jnp.pad(edge_index, ((0, 0), (0, EPAD - E)), constant_values=N)
    src2d = ei[0].reshape(EPAD // CH, CH)
    dst2d = ei[1].reshape(EPAD // CH, CH)

    ht1, ast1, adt1 = _prep1(xp, W1, a_src1, a_dst1)
    out1 = _sc_layer1(src2d, dst2d, ast1, adt1,
                      ht1.reshape(HEADS * NPAD, HID))
    ht2, ast2, adt2 = _prep2(out1, b1.reshape(HEADS, HID),
                             W2.reshape(HEADS, HID, HEADS * C),
                             a_src2, a_dst2)
    part = _sc_layer2(src2d, dst2d, ast2, adt2,
                      ht2.reshape(HEADS * NPAD, C))
    out = _final(part, b2.reshape(1, C))
    return out[:N]


# speedup vs baseline: 18.7540x; 1.2762x over previous
import functools

import jax
import jax.numpy as jnp
from jax import lax
from jax.experimental import pallas as pl
from jax.experimental.pallas import tpu as pltpu
from jax.experimental.pallas import tpu_sc as plsc

N = 10000
E = 320000
F_IN = 128
HID = 64
HEADS = 8
C = 64

NPAD = 10240
EPAD = 327680
NC, NS, L = 2, 16, 16
SL = NPAD // NS
EPT = EPAD // NS
CH = 128
SBLK = 2048
NBLK = EPT // SBLK
BCH = SBLK // CH
HPC = HEADS // NC
NEG_SLOPE = 0.2

f32 = jnp.float32
i32 = jnp.int32


def _prep1_body(x_ref, w_ref, asr_ref, adr_ref, ht_ref, ast_ref, adt_ref):
    h = jnp.dot(x_ref[...], w_ref[...], preferred_element_type=f32)
    for hh in range(HEADS):
        blk = h[:, hh * HID:(hh + 1) * HID]
        ht_ref[hh] = blk
        ast_ref[hh] = jnp.sum(blk * asr_ref[hh][None, :], axis=1)
        adt_ref[hh] = jnp.sum(blk * adr_ref[hh][None, :], axis=1)


def _prep1(xp, W1, a_src, a_dst):
    BR = 1024
    nb = NPAD // BR
    return pl.pallas_call(
        _prep1_body,
        grid=(nb,),
        in_specs=[
            pl.BlockSpec((BR, F_IN), lambda i: (i, 0)),
            pl.BlockSpec((F_IN, HEADS * HID), lambda i: (0, 0)),
            pl.BlockSpec((HEADS, HID), lambda i: (0, 0)),
            pl.BlockSpec((HEADS, HID), lambda i: (0, 0)),
        ],
        out_specs=[
            pl.BlockSpec((HEADS, BR, HID), lambda i: (0, i, 0)),
            pl.BlockSpec((HEADS, BR), lambda i: (0, i)),
            pl.BlockSpec((HEADS, BR), lambda i: (0, i)),
        ],
        out_shape=[
            jax.ShapeDtypeStruct((HEADS, NPAD, HID), f32),
            jax.ShapeDtypeStruct((HEADS, NPAD), f32),
            jax.ShapeDtypeStruct((HEADS, NPAD), f32),
        ],
    )(xp, W1, a_src, a_dst)


def _prep2_body(o1_ref, b1_ref, w2_ref, asr_ref, adr_ref,
                ht_ref, ast_ref, adt_ref):
    acc = jnp.zeros((o1_ref.shape[1], HEADS * C), f32)
    for hh in range(HEADS):
        v = o1_ref[hh] + b1_ref[hh][None, :]
        v = jnp.where(v > 0, v, jnp.exp(v) - 1.0)
        acc = acc + jnp.dot(v, w2_ref[hh], preferred_element_type=f32)
    for hh in range(HEADS):
        blk = acc[:, hh * C:(hh + 1) * C]
        ht_ref[hh] = blk
        ast_ref[hh] = jnp.sum(blk * asr_ref[hh][None, :], axis=1)
        adt_ref[hh] = jnp.sum(blk * adr_ref[hh][None, :], axis=1)


def _prep2(out1, b1r, W2r, a_src, a_dst):
    BR = 1024
    nb = NPAD // BR
    return pl.pallas_call(
        _prep2_body,
        grid=(nb,),
        in_specs=[
            pl.BlockSpec((HEADS, BR, HID), lambda i: (0, i, 0)),
            pl.BlockSpec((HEADS, HID), lambda i: (0, 0)),
            pl.BlockSpec((HEADS, HID, HEADS * C), lambda i: (0, 0, 0)),
            pl.BlockSpec((HEADS, C), lambda i: (0, 0)),
            pl.BlockSpec((HEADS, C), lambda i: (0, 0)),
        ],
        out_specs=[
            pl.BlockSpec((HEADS, BR, C), lambda i: (0, i, 0)),
            pl.BlockSpec((HEADS, BR), lambda i: (0, i)),
            pl.BlockSpec((HEADS, BR), lambda i: (0, i)),
        ],
        out_shape=[
            jax.ShapeDtypeStruct((HEADS, NPAD, C), f32),
            jax.ShapeDtypeStruct((HEADS, NPAD), f32),
            jax.ShapeDtypeStruct((HEADS, NPAD), f32),
        ],
    )(out1, b1r, W2r, a_src, a_dst)


def _final_body(p_ref, b2_ref, o_ref):
    o_ref[...] = p_ref[0] + p_ref[1] + b2_ref[...]


def _final(part, b2r):
    BR = 1000
    nb = N // BR
    return pl.pallas_call(
        _final_body,
        grid=(nb,),
        in_specs=[
            pl.BlockSpec((2, BR, C), lambda i: (0, i, 0)),
            pl.BlockSpec((1, C), lambda i: (0, 0)),
        ],
        out_specs=pl.BlockSpec((BR, C), lambda i: (i, 0)),
        out_shape=jax.ShapeDtypeStruct((N, C), f32),
    )(part, b2r)


def _pad_x_body(x_ref, o_ref):
    o_ref[pl.ds(0, N), :] = x_ref[...]
    o_ref[pl.ds(N, NPAD - N), :] = jnp.zeros((NPAD - N, F_IN), f32)


def _pad_x(x):
    return pl.pallas_call(
        _pad_x_body,
        out_shape=jax.ShapeDtypeStruct((NPAD, F_IN), f32),
    )(x)


def _pad_edges_body(e_ref, s_ref, d_ref):
    fill = jnp.full((EPAD // CH - E // CH, CH), N, dtype=i32)
    s_ref[pl.ds(0, E // CH), :] = e_ref[0]
    s_ref[pl.ds(E // CH, EPAD // CH - E // CH), :] = fill
    d_ref[pl.ds(0, E // CH), :] = e_ref[1]
    d_ref[pl.ds(E // CH, EPAD // CH - E // CH), :] = fill


def _pad_edges(ei3):
    return pl.pallas_call(
        _pad_edges_body,
        out_shape=[
            jax.ShapeDtypeStruct((EPAD // CH, CH), i32),
            jax.ShapeDtypeStruct((EPAD // CH, CH), i32),
        ],
    )(ei3)


def _make_sc_layer(concat):
    scale = 1.0 if concat else 1.0 / HEADS
    out_shape = (jax.ShapeDtypeStruct((HEADS, NPAD, HID), f32) if concat
                 else jax.ShapeDtypeStruct((NC, NPAD, C), f32))
    mesh = plsc.VectorSubcoreMesh(core_axis_name="c", subcore_axis_name="s",
                                  num_cores=NC, num_subcores=NS)

    @functools.partial(
        pl.kernel, mesh=mesh, out_type=out_shape,
        compiler_params=pltpu.CompilerParams(needs_layout_passes=False,
                                             use_tc_tiling_on_sc=False),
        scratch_types=[
            pltpu.VMEM((BCH, CH), i32),
            pltpu.VMEM((BCH, CH), i32),
            pltpu.VMEM((2, CH), i32),
            pltpu.VMEM((2, CH), f32),
            pltpu.VMEM((NPAD,), f32),
            pltpu.VMEM((NPAD,), f32),
            pltpu.VMEM((NPAD,), f32),
            pltpu.VMEM((NS // 2, SL), f32),
            pltpu.VMEM((SL,), f32),
            pltpu.VMEM((CH, HID), f32),
            pltpu.VMEM((CH, HID), f32),
            pltpu.VMEM_SHARED((NPAD, HID), f32),
            pltpu.VMEM_SHARED((NS, NPAD), f32),
            pltpu.VMEM_SHARED((NPAD,), f32),
            pltpu.SemaphoreType.DMA,
            pltpu.SemaphoreType.DMA,
        ],
    )
    def sck(src_hbm, dst_hbm, ast_hbm, adt_hbm, ht_hbm, z_hbm, out_hbm,
            src_v, dst_v, adj_v, cf_v, as_v, ad_v, den_v, mrg_v, red_v,
            rows0_v, rows1_v, out_sp, dsp, mer, sem0, sem1):
        rows = (rows0_v, rows1_v)
        sems = (sem0, sem1)
        c = lax.axis_index("c")
        s = lax.axis_index("s")
        zero16 = jnp.zeros((L,), f32)

        def zero_out_slice():
            for k in range(SL // CH):
                pltpu.sync_copy(z_hbm,
                                out_sp.at[pl.ds(s * SL + k * CH, CH), :])
        zero_out_slice()

        def stage(b):
            row0 = s * (EPT // CH) + b * BCH
            pltpu.sync_copy(src_hbm.at[pl.ds(row0, BCH), :], src_v)
            pltpu.sync_copy(dst_hbm.at[pl.ds(row0, BCH), :], dst_v)

        def edge_e(s16, d16):
            a = plsc.load_gather(as_v, [s16]) + plsc.load_gather(ad_v, [d16])
            a = jnp.where(a > 0, a, NEG_SLOPE * a)
            return jnp.exp(a)

        def head_step(hh, _):
            head = c * HPC + hh

            def dz(i, _):
                den_v[pl.ds(i * L, L)] = zero16
                return 0
            lax.fori_loop(0, NPAD // L, dz, 0)
            pltpu.sync_copy(ast_hbm.at[head], as_v)
            pltpu.sync_copy(adt_hbm.at[head], ad_v)

            def p1_block(b, _):
                stage(b)

                def p1_chunk(j, _):
                    for v in range(CH // L):
                        sl = pl.ds(v * L, L)
                        d16 = dst_v[j, sl]
                        plsc.addupdate_scatter(
                            den_v, [d16], edge_e(src_v[j, sl], d16))
                    return 0
                lax.fori_loop(0, BCH, p1_chunk, 0)
                return 0
            lax.fori_loop(0, NBLK, p1_block, 0)

            pltpu.sync_copy(den_v, dsp.at[s])
            plsc.subcore_barrier()
            col = pl.ds(s * SL, SL)
            for half in range(2):
                pltpu.sync_copy(
                    dsp.at[pl.ds(half * (NS // 2), NS // 2), col], mrg_v)

                def dred(v2, _):
                    sl = pl.ds(v2 * L, L)
                    acc = (mrg_v[0, sl] if half == 0
                           else red_v[sl] + mrg_v[0, sl])
                    for r in range(1, NS // 2):
                        acc = acc + mrg_v[r, sl]
                    red_v[sl] = acc
                    return 0
                lax.fori_loop(0, SL // L, dred, 0)
            pltpu.sync_copy(red_v, mer.at[col])
            plsc.subcore_barrier()
            pltpu.sync_copy(mer, den_v)

            def drcp(i, _):
                sl = pl.ds(i * L, L)
                den_v[sl] = scale / (den_v[sl] + 1e-16)
                return 0
            lax.fori_loop(0, NPAD // L, drcp, 0)

            off = head * NPAD

            def coef_chunk(j, bb):
                for v in range(CH // L):
                    sl = pl.ds(v * L, L)
                    s16 = src_v[j, sl]
                    d16 = dst_v[j, sl]
                    rcp = plsc.load_gather(den_v, [d16])
                    cf_v[bb, sl] = edge_e(s16, d16) * rcp
                    adj_v[bb, sl] = s16 + off

            def fire_gather(bb):
                pltpu.async_copy(ht_hbm.at[adj_v.at[bb]], rows[bb], sems[bb])

            def wait_gather(bb):
                pltpu.make_async_copy(ht_hbm.at[pl.ds(0, CH), :],
                                      rows[bb], sems[bb]).wait()

            def scale_scatter(j, bb):
                rv = rows[bb]

                def rscale(i2, _):
                    for u in range(2):
                        i = i2 * 2 + u
                        bc = plsc.load_gather(
                            cf_v.at[bb], [jnp.full((L,), i, dtype=i32)])
                        for q in range(HID // L):
                            sq = pl.ds(q * L, L)
                            rv[i, sq] = rv[i, sq] * bc
                    return 0
                lax.fori_loop(0, CH // 2, rscale, 0)
                pltpu.sync_copy(rv, out_sp.at[dst_v.at[j]], add=True)

            def p2_block(b, _):
                stage(b)
                coef_chunk(0, 0)
                fire_gather(0)

                def p2_pair(j2, _):
                    for bb in range(2):
                        j = 2 * j2 + bb
                        wait_gather(bb)
                        if bb == 0:
                            coef_chunk(j + 1, 1)
                            fire_gather(1)
                        else:
                            @pl.when(j2 < BCH // 2 - 1)
                            def _():
                                coef_chunk(j + 1, 0)
                                fire_gather(0)
                        scale_scatter(j, bb)
                    return 0
                lax.fori_loop(0, BCH // 2, p2_pair, 0)
                return 0
            lax.fori_loop(0, NBLK, p2_block, 0)

            if concat:
                plsc.subcore_barrier()
                pltpu.sync_copy(out_sp.at[pl.ds(s * SL, SL), :],
                                out_hbm.at[head, pl.ds(s * SL, SL), :])
                zero_out_slice()
            return 0

        lax.fori_loop(0, HPC, head_step, 0)

        if not concat:
            plsc.subcore_barrier()
            pltpu.sync_copy(out_sp.at[pl.ds(s * SL, SL), :],
                            out_hbm.at[c, pl.ds(s * SL, SL), :])

    return sck


_sc_layer1 = _make_sc_layer(concat=True)
_sc_layer2 = _make_sc_layer(concat=False)


def kernel(x, edge_index, W1, a_src1, a_dst1, b1, W2, a_src2, a_dst2, b2):
    xp = _pad_x(x)
    src2d, dst2d = _pad_edges(edge_index.reshape(2, E // CH, CH))
    z64 = jnp.zeros((CH, HID), f32)

    ht1, ast1, adt1 = _prep1(xp, W1, a_src1, a_dst1)
    out1 = _sc_layer1(src2d, dst2d, ast1, adt1,
                      ht1.reshape(HEADS * NPAD, HID), z64)
    ht2, ast2, adt2 = _prep2(out1, b1.reshape(HEADS, HID),
                             W2.reshape(HEADS, HID, HEADS * C),
                             a_src2, a_dst2)
    part = _sc_layer2(src2d, dst2d, ast2, adt2,
                      ht2.reshape(HEADS * NPAD, C), z64)
    return _final(part, b2.reshape(1, C))


# speedup vs baseline: 20.0413x; 1.0686x over previous
import functools

import jax
import jax.numpy as jnp
from jax import lax
from jax.experimental import pallas as pl
from jax.experimental.pallas import tpu as pltpu
from jax.experimental.pallas import tpu_sc as plsc

N = 10000
E = 320000
F_IN = 128
HID = 64
HEADS = 8
C = 64

NPAD = 10240
EPAD = 327680
NC, NS, L = 2, 16, 16
SL = NPAD // NS
EPT = EPAD // NS
CH = 128
SBLK = 2048
NBLK = EPT // SBLK
BCH = SBLK // CH
HPC = HEADS // NC
NEG_SLOPE = 0.2

f32 = jnp.float32
i32 = jnp.int32


def _prep1_body(x_ref, w_ref, asr_ref, adr_ref, ht_ref, ast_ref, adt_ref):
    h = jnp.dot(x_ref[...], w_ref[...], preferred_element_type=f32)
    for hh in range(HEADS):
        blk = h[:, hh * HID:(hh + 1) * HID]
        ht_ref[hh] = blk
        ast_ref[hh] = jnp.sum(blk * asr_ref[hh][None, :], axis=1)
        adt_ref[hh] = jnp.sum(blk * adr_ref[hh][None, :], axis=1)


def _prep1(xp, W1, a_src, a_dst):
    BR = 1024
    nb = NPAD // BR
    return pl.pallas_call(
        _prep1_body,
        grid=(nb,),
        in_specs=[
            pl.BlockSpec((BR, F_IN), lambda i: (i, 0)),
            pl.BlockSpec((F_IN, HEADS * HID), lambda i: (0, 0)),
            pl.BlockSpec((HEADS, HID), lambda i: (0, 0)),
            pl.BlockSpec((HEADS, HID), lambda i: (0, 0)),
        ],
        out_specs=[
            pl.BlockSpec((HEADS, BR, HID), lambda i: (0, i, 0)),
            pl.BlockSpec((HEADS, BR), lambda i: (0, i)),
            pl.BlockSpec((HEADS, BR), lambda i: (0, i)),
        ],
        out_shape=[
            jax.ShapeDtypeStruct((HEADS, NPAD, HID), f32),
            jax.ShapeDtypeStruct((HEADS, NPAD), f32),
            jax.ShapeDtypeStruct((HEADS, NPAD), f32),
        ],
    )(xp, W1, a_src, a_dst)


def _prep2_body(o1_ref, b1_ref, w2_ref, asr_ref, adr_ref,
                ht_ref, ast_ref, adt_ref):
    acc = jnp.zeros((o1_ref.shape[1], HEADS * C), f32)
    for hh in range(HEADS):
        v = o1_ref[hh] + b1_ref[hh][None, :]
        v = jnp.where(v > 0, v, jnp.exp(v) - 1.0)
        acc = acc + jnp.dot(v, w2_ref[hh], preferred_element_type=f32)
    for hh in range(HEADS):
        blk = acc[:, hh * C:(hh + 1) * C]
        ht_ref[hh] = blk
        ast_ref[hh] = jnp.sum(blk * asr_ref[hh][None, :], axis=1)
        adt_ref[hh] = jnp.sum(blk * adr_ref[hh][None, :], axis=1)


def _prep2(out1, b1r, W2r, a_src, a_dst):
    BR = 1024
    nb = NPAD // BR
    return pl.pallas_call(
        _prep2_body,
        grid=(nb,),
        in_specs=[
            pl.BlockSpec((HEADS, BR, HID), lambda i: (0, i, 0)),
            pl.BlockSpec((HEADS, HID), lambda i: (0, 0)),
            pl.BlockSpec((HEADS, HID, HEADS * C), lambda i: (0, 0, 0)),
            pl.BlockSpec((HEADS, C), lambda i: (0, 0)),
            pl.BlockSpec((HEADS, C), lambda i: (0, 0)),
        ],
        out_specs=[
            pl.BlockSpec((HEADS, BR, C), lambda i: (0, i, 0)),
            pl.BlockSpec((HEADS, BR), lambda i: (0, i)),
            pl.BlockSpec((HEADS, BR), lambda i: (0, i)),
        ],
        out_shape=[
            jax.ShapeDtypeStruct((HEADS, NPAD, C), f32),
            jax.ShapeDtypeStruct((HEADS, NPAD), f32),
            jax.ShapeDtypeStruct((HEADS, NPAD), f32),
        ],
    )(out1, b1r, W2r, a_src, a_dst)


def _final_body(p_ref, b2_ref, o_ref):
    o_ref[...] = p_ref[0] + p_ref[1] + b2_ref[...]


def _final(part, b2r):
    BR = 1000
    nb = N // BR
    return pl.pallas_call(
        _final_body,
        grid=(nb,),
        in_specs=[
            pl.BlockSpec((2, BR, C), lambda i: (0, i, 0)),
            pl.BlockSpec((1, C), lambda i: (0, 0)),
        ],
        out_specs=pl.BlockSpec((BR, C), lambda i: (i, 0)),
        out_shape=jax.ShapeDtypeStruct((N, C), f32),
    )(part, b2r)


def _pad_x_body(x_ref, o_ref):
    o_ref[pl.ds(0, N), :] = x_ref[...]
    o_ref[pl.ds(N, NPAD - N), :] = jnp.zeros((NPAD - N, F_IN), f32)


def _pad_x(x):
    return pl.pallas_call(
        _pad_x_body,
        out_shape=jax.ShapeDtypeStruct((NPAD, F_IN), f32),
    )(x)


def _pad_edges_body(e_ref, s_ref, d_ref):
    fill = jnp.full((EPAD // CH - E // CH, CH), N, dtype=i32)
    s_ref[pl.ds(0, E // CH), :] = e_ref[0]
    s_ref[pl.ds(E // CH, EPAD // CH - E // CH), :] = fill
    d_ref[pl.ds(0, E // CH), :] = e_ref[1]
    d_ref[pl.ds(E // CH, EPAD // CH - E // CH), :] = fill


def _pad_edges(ei3):
    return pl.pallas_call(
        _pad_edges_body,
        out_shape=[
            jax.ShapeDtypeStruct((EPAD // CH, CH), i32),
            jax.ShapeDtypeStruct((EPAD // CH, CH), i32),
        ],
    )(ei3)


def _make_sc_layer(concat):
    scale = 1.0 if concat else 1.0 / HEADS
    out_shape = (jax.ShapeDtypeStruct((HEADS, NPAD, HID), f32) if concat
                 else jax.ShapeDtypeStruct((NC, NPAD, C), f32))
    mesh = plsc.VectorSubcoreMesh(core_axis_name="c", subcore_axis_name="s",
                                  num_cores=NC, num_subcores=NS)

    @functools.partial(
        pl.kernel, mesh=mesh, out_type=out_shape,
        compiler_params=pltpu.CompilerParams(needs_layout_passes=False,
                                             use_tc_tiling_on_sc=False),
        scratch_types=[
            pltpu.VMEM((BCH, CH), i32),
            pltpu.VMEM((BCH, CH), i32),
            pltpu.VMEM((4, CH), i32),
            pltpu.VMEM((4, CH), f32),
            pltpu.VMEM((NPAD,), f32),
            pltpu.VMEM((NPAD,), f32),
            pltpu.VMEM((NPAD,), f32),
            pltpu.VMEM((NS // 4, SL), f32),
            pltpu.VMEM((SL,), f32),
            pltpu.VMEM((CH, HID), f32),
            pltpu.VMEM((CH, HID), f32),
            pltpu.VMEM((CH, HID), f32),
            pltpu.VMEM((CH, HID), f32),
            pltpu.VMEM_SHARED((NPAD, HID), f32),
            pltpu.VMEM_SHARED((NS, NPAD), f32),
            pltpu.VMEM_SHARED((NPAD,), f32),
            pltpu.SemaphoreType.DMA,
            pltpu.SemaphoreType.DMA,
            pltpu.SemaphoreType.DMA,
            pltpu.SemaphoreType.DMA,
        ],
    )
    def sck(src_hbm, dst_hbm, ast_hbm, adt_hbm, ht_hbm, z_hbm, out_hbm,
            src_v, dst_v, adj_v, cf_v, as_v, ad_v, den_v, mrg_v, red_v,
            rows0_v, rows1_v, rows2_v, rows3_v, out_sp, dsp, mer,
            sem0, sem1, sem2, sem3):
        rows = (rows0_v, rows1_v, rows2_v, rows3_v)
        sems = (sem0, sem1, sem2, sem3)
        c = lax.axis_index("c")
        s = lax.axis_index("s")
        zero16 = jnp.zeros((L,), f32)

        def zero_out_slice():
            for k in range(SL // CH):
                pltpu.sync_copy(z_hbm,
                                out_sp.at[pl.ds(s * SL + k * CH, CH), :])
        zero_out_slice()

        def stage(b):
            row0 = s * (EPT // CH) + b * BCH
            pltpu.sync_copy(src_hbm.at[pl.ds(row0, BCH), :], src_v)
            pltpu.sync_copy(dst_hbm.at[pl.ds(row0, BCH), :], dst_v)

        def edge_e(s16, d16):
            a = plsc.load_gather(as_v, [s16]) + plsc.load_gather(ad_v, [d16])
            a = jnp.where(a > 0, a, NEG_SLOPE * a)
            return jnp.exp(a)

        def head_step(hh, _):
            head = c * HPC + hh

            def dz(i, _):
                den_v[pl.ds(i * L, L)] = zero16
                return 0
            lax.fori_loop(0, NPAD // L, dz, 0)
            pltpu.sync_copy(ast_hbm.at[head], as_v)
            pltpu.sync_copy(adt_hbm.at[head], ad_v)

            def p1_block(b, _):
                stage(b)

                def p1_chunk(j, _):
                    for v in range(CH // L):
                        sl = pl.ds(v * L, L)
                        d16 = dst_v[j, sl]
                        plsc.addupdate_scatter(
                            den_v, [d16], edge_e(src_v[j, sl], d16))
                    return 0
                lax.fori_loop(0, BCH, p1_chunk, 0)
                return 0
            lax.fori_loop(0, NBLK, p1_block, 0)

            pltpu.sync_copy(den_v, dsp.at[s])
            plsc.subcore_barrier()
            col = pl.ds(s * SL, SL)
            for quarter in range(4):
                pltpu.sync_copy(
                    dsp.at[pl.ds(quarter * (NS // 4), NS // 4), col], mrg_v)

                def dred(v2, _):
                    sl = pl.ds(v2 * L, L)
                    acc = (mrg_v[0, sl] if quarter == 0
                           else red_v[sl] + mrg_v[0, sl])
                    for r in range(1, NS // 4):
                        acc = acc + mrg_v[r, sl]
                    red_v[sl] = acc
                    return 0
                lax.fori_loop(0, SL // L, dred, 0)
            pltpu.sync_copy(red_v, mer.at[col])
            plsc.subcore_barrier()
            pltpu.sync_copy(mer, den_v)

            def drcp(i, _):
                sl = pl.ds(i * L, L)
                den_v[sl] = scale / (den_v[sl] + 1e-16)
                return 0
            lax.fori_loop(0, NPAD // L, drcp, 0)

            off = head * NPAD

            def coef_chunk(j, bb):
                for v in range(CH // L):
                    sl = pl.ds(v * L, L)
                    s16 = src_v[j, sl]
                    d16 = dst_v[j, sl]
                    rcp = plsc.load_gather(den_v, [d16])
                    cf_v[bb, sl] = edge_e(s16, d16) * rcp
                    adj_v[bb, sl] = s16 + off

            def fire_gather(bb):
                pltpu.async_copy(ht_hbm.at[adj_v.at[bb]], rows[bb], sems[bb])

            def wait_gather(bb):
                pltpu.make_async_copy(ht_hbm.at[pl.ds(0, CH), :],
                                      rows[bb], sems[bb]).wait()

            def scale_scatter(j, bb):
                rv = rows[bb]

                def rscale(i2, _):
                    for u in range(2):
                        i = i2 * 2 + u
                        bc = plsc.load_gather(
                            cf_v.at[bb], [jnp.full((L,), i, dtype=i32)])
                        for q in range(HID // L):
                            sq = pl.ds(q * L, L)
                            rv[i, sq] = rv[i, sq] * bc
                    return 0
                lax.fori_loop(0, CH // 2, rscale, 0)
                pltpu.sync_copy(rv, out_sp.at[dst_v.at[j]], add=True)

            def p2_block(b, _):
                stage(b)
                for bb in range(3):
                    coef_chunk(bb, bb)
                    fire_gather(bb)

                def p2_quad(j4, _):
                    for u in range(4):
                        j = 4 * j4 + u
                        bb = u
                        wait_gather(bb)
                        nb = (u + 3) % 4
                        if u == 0:
                            coef_chunk(j + 3, nb)
                            fire_gather(nb)
                        else:
                            @pl.when(j4 < BCH // 4 - 1)
                            def _():
                                coef_chunk(j + 3, nb)
                                fire_gather(nb)
                        scale_scatter(j, bb)
                    return 0
                lax.fori_loop(0, BCH // 4, p2_quad, 0)
                return 0
            lax.fori_loop(0, NBLK, p2_block, 0)

            if concat:
                plsc.subcore_barrier()
                pltpu.sync_copy(out_sp.at[pl.ds(s * SL, SL), :],
                                out_hbm.at[head, pl.ds(s * SL, SL), :])
                zero_out_slice()
            return 0

        lax.fori_loop(0, HPC, head_step, 0)

        if not concat:
            plsc.subcore_barrier()
            pltpu.sync_copy(out_sp.at[pl.ds(s * SL, SL), :],
                            out_hbm.at[c, pl.ds(s * SL, SL), :])

    return sck


_sc_layer1 = _make_sc_layer(concat=True)
_sc_layer2 = _make_sc_layer(concat=False)


def kernel(x, edge_index, W1, a_src1, a_dst1, b1, W2, a_src2, a_dst2, b2):
    xp = _pad_x(x)
    src2d, dst2d = _pad_edges(edge_index.reshape(2, E // CH, CH))
    z64 = jnp.zeros((CH, HID), f32)

    ht1, ast1, adt1 = _prep1(xp, W1, a_src1, a_dst1)
    out1 = _sc_layer1(src2d, dst2d, ast1, adt1,
                      ht1.reshape(HEADS * NPAD, HID), z64)
    ht2, ast2, adt2 = _prep2(out1, b1.reshape(HEADS, HID),
                             W2.reshape(HEADS, HID, HEADS * C),
                             a_src2, a_dst2)
    part = _sc_layer2(src2d, dst2d, ast2, adt2,
                      ht2.reshape(HEADS * NPAD, C), z64)
    return _final(part, b2.reshape(1, C))


# speedup vs baseline: 20.7999x; 1.0379x over previous
import functools

import jax
import jax.numpy as jnp
from jax import lax
from jax.experimental import pallas as pl
from jax.experimental.pallas import tpu as pltpu
from jax.experimental.pallas import tpu_sc as plsc

N = 10000
E = 320000
F_IN = 128
HID = 64
HEADS = 8
C = 64

NPAD = 10240
EPAD = 327680
NC, NS, L = 2, 16, 16
SL = NPAD // NS
EPT = EPAD // NS
CH = 128
SBLK = 2048
NBLK = EPT // SBLK
BCH = SBLK // CH
HPC = HEADS // NC
NEG_SLOPE = 0.2

f32 = jnp.float32
i32 = jnp.int32


def _pack_rows(blk):
    parts = []
    for g in range(blk.shape[1] // 32):
        a = blk[:, 32 * g:32 * g + 16].astype(jnp.bfloat16)
        b = blk[:, 32 * g + 16:32 * g + 32].astype(jnp.bfloat16)
        ai = lax.bitcast_convert_type(a, jnp.uint16).astype(i32)
        bi = lax.bitcast_convert_type(b, jnp.uint16).astype(i32)
        parts.append(ai | (bi << 16))
    return jnp.concatenate(parts, axis=1)


def _prep1_body(x_ref, w_ref, asr_ref, adr_ref, tb_ref, ast_ref, adt_ref):
    h = jnp.dot(x_ref[...], w_ref[...], preferred_element_type=f32)
    for hh in range(HEADS):
        blk = h[:, hh * HID:(hh + 1) * HID]
        tb_ref[hh] = _pack_rows(blk)
        ast_ref[hh] = jnp.sum(blk * asr_ref[hh][None, :], axis=1)
        adt_ref[hh] = jnp.sum(blk * adr_ref[hh][None, :], axis=1)


def _prep1(xp, W1, a_src, a_dst):
    BR = 1024
    nb = NPAD // BR
    return pl.pallas_call(
        _prep1_body,
        grid=(nb,),
        in_specs=[
            pl.BlockSpec((BR, F_IN), lambda i: (i, 0)),
            pl.BlockSpec((F_IN, HEADS * HID), lambda i: (0, 0)),
            pl.BlockSpec((HEADS, HID), lambda i: (0, 0)),
            pl.BlockSpec((HEADS, HID), lambda i: (0, 0)),
        ],
        out_specs=[
            pl.BlockSpec((HEADS, BR, HID // 2), lambda i: (0, i, 0)),
            pl.BlockSpec((HEADS, BR), lambda i: (0, i)),
            pl.BlockSpec((HEADS, BR), lambda i: (0, i)),
        ],
        out_shape=[
            jax.ShapeDtypeStruct((HEADS, NPAD, HID // 2), i32),
            jax.ShapeDtypeStruct((HEADS, NPAD), f32),
            jax.ShapeDtypeStruct((HEADS, NPAD), f32),
        ],
    )(xp, W1, a_src, a_dst)


def _prep2_body(o1_ref, b1_ref, w2_ref, asr_ref, adr_ref,
                ht_ref, ast_ref, adt_ref):
    acc = jnp.zeros((o1_ref.shape[1], HEADS * C), f32)
    for hh in range(HEADS):
        v = o1_ref[hh] + b1_ref[hh][None, :]
        v = jnp.where(v > 0, v, jnp.exp(v) - 1.0)
        acc = acc + jnp.dot(v, w2_ref[hh], preferred_element_type=f32)
    for hh in range(HEADS):
        blk = acc[:, hh * C:(hh + 1) * C]
        ht_ref[hh] = _pack_rows(blk)
        ast_ref[hh] = jnp.sum(blk * asr_ref[hh][None, :], axis=1)
        adt_ref[hh] = jnp.sum(blk * adr_ref[hh][None, :], axis=1)


def _prep2(out1, b1r, W2r, a_src, a_dst):
    BR = 1024
    nb = NPAD // BR
    return pl.pallas_call(
        _prep2_body,
        grid=(nb,),
        in_specs=[
            pl.BlockSpec((HEADS, BR, HID), lambda i: (0, i, 0)),
            pl.BlockSpec((HEADS, HID), lambda i: (0, 0)),
            pl.BlockSpec((HEADS, HID, HEADS * C), lambda i: (0, 0, 0)),
            pl.BlockSpec((HEADS, C), lambda i: (0, 0)),
            pl.BlockSpec((HEADS, C), lambda i: (0, 0)),
        ],
        out_specs=[
            pl.BlockSpec((HEADS, BR, C // 2), lambda i: (0, i, 0)),
            pl.BlockSpec((HEADS, BR), lambda i: (0, i)),
            pl.BlockSpec((HEADS, BR), lambda i: (0, i)),
        ],
        out_shape=[
            jax.ShapeDtypeStruct((HEADS, NPAD, C // 2), i32),
            jax.ShapeDtypeStruct((HEADS, NPAD), f32),
            jax.ShapeDtypeStruct((HEADS, NPAD), f32),
        ],
    )(out1, b1r, W2r, a_src, a_dst)


def _final_body(p_ref, b2_ref, o_ref):
    o_ref[...] = p_ref[0] + p_ref[1] + b2_ref[...]


def _final(part, b2r):
    BR = 1000
    nb = N // BR
    return pl.pallas_call(
        _final_body,
        grid=(nb,),
        in_specs=[
            pl.BlockSpec((2, BR, C), lambda i: (0, i, 0)),
            pl.BlockSpec((1, C), lambda i: (0, 0)),
        ],
        out_specs=pl.BlockSpec((BR, C), lambda i: (i, 0)),
        out_shape=jax.ShapeDtypeStruct((N, C), f32),
    )(part, b2r)


def _pad_x_body(x_ref, o_ref):
    o_ref[pl.ds(0, N), :] = x_ref[...]
    o_ref[pl.ds(N, NPAD - N), :] = jnp.zeros((NPAD - N, F_IN), f32)


def _pad_x(x):
    return pl.pallas_call(
        _pad_x_body,
        out_shape=jax.ShapeDtypeStruct((NPAD, F_IN), f32),
    )(x)


def _pad_edges_body(e_ref, s_ref, d_ref):
    fill = jnp.full((EPAD // CH - E // CH, CH), N, dtype=i32)
    s_ref[pl.ds(0, E // CH), :] = e_ref[0]
    s_ref[pl.ds(E // CH, EPAD // CH - E // CH), :] = fill
    d_ref[pl.ds(0, E // CH), :] = e_ref[1]
    d_ref[pl.ds(E // CH, EPAD // CH - E // CH), :] = fill


def _pad_edges(ei3):
    return pl.pallas_call(
        _pad_edges_body,
        out_shape=[
            jax.ShapeDtypeStruct((EPAD // CH, CH), i32),
            jax.ShapeDtypeStruct((EPAD // CH, CH), i32),
        ],
    )(ei3)


def _make_sc_layer(concat):
    scale = 1.0 if concat else 1.0 / HEADS
    out_shape = (jax.ShapeDtypeStruct((HEADS, NPAD, HID), f32) if concat
                 else jax.ShapeDtypeStruct((NC, NPAD, C), f32))
    mesh = plsc.VectorSubcoreMesh(core_axis_name="c", subcore_axis_name="s",
                                  num_cores=NC, num_subcores=NS)

    @functools.partial(
        pl.kernel, mesh=mesh, out_type=out_shape,
        compiler_params=pltpu.CompilerParams(needs_layout_passes=False,
                                             use_tc_tiling_on_sc=False),
        scratch_types=[
            pltpu.VMEM((BCH, CH), i32),
            pltpu.VMEM((BCH, CH), i32),
            pltpu.VMEM((4, CH), i32),
            pltpu.VMEM((4, CH), f32),
            pltpu.VMEM((NPAD,), f32),
            pltpu.VMEM((NPAD,), f32),
            pltpu.VMEM((NPAD,), f32),
            pltpu.VMEM((NS // 4, SL), f32),
            pltpu.VMEM((SL,), f32),
            pltpu.VMEM((CH, HID // 2), i32),
            pltpu.VMEM((CH, HID // 2), i32),
            pltpu.VMEM((CH, HID // 2), i32),
            pltpu.VMEM((CH, HID // 2), i32),
            pltpu.VMEM((CH, HID), f32),
            pltpu.VMEM_SHARED((NPAD, HID), f32),
            pltpu.VMEM_SHARED((NS, NPAD), f32),
            pltpu.VMEM_SHARED((NPAD,), f32),
            pltpu.SemaphoreType.DMA,
            pltpu.SemaphoreType.DMA,
            pltpu.SemaphoreType.DMA,
            pltpu.SemaphoreType.DMA,
        ],
    )
    def sck(src_hbm, dst_hbm, ast_hbm, adt_hbm, ht_hbm, z_hbm, out_hbm,
            src_v, dst_v, adj_v, cf_v, as_v, ad_v, den_v, mrg_v, red_v,
            rows0_v, rows1_v, rows2_v, rows3_v, rowf_v, out_sp, dsp, mer,
            sem0, sem1, sem2, sem3):
        rows = (rows0_v, rows1_v, rows2_v, rows3_v)
        sems = (sem0, sem1, sem2, sem3)
        c = lax.axis_index("c")
        s = lax.axis_index("s")
        zero16 = jnp.zeros((L,), f32)

        def zero_out_slice():
            for k in range(SL // CH):
                pltpu.sync_copy(z_hbm,
                                out_sp.at[pl.ds(s * SL + k * CH, CH), :])
        zero_out_slice()

        def stage(b):
            row0 = s * (EPT // CH) + b * BCH
            pltpu.sync_copy(src_hbm.at[pl.ds(row0, BCH), :], src_v)
            pltpu.sync_copy(dst_hbm.at[pl.ds(row0, BCH), :], dst_v)

        def edge_e(s16, d16):
            a = plsc.load_gather(as_v, [s16]) + plsc.load_gather(ad_v, [d16])
            a = jnp.where(a > 0, a, NEG_SLOPE * a)
            return jnp.exp(a)

        def head_step(hh, _):
            head = c * HPC + hh

            def dz(i, _):
                den_v[pl.ds(i * L, L)] = zero16
                return 0
            lax.fori_loop(0, NPAD // L, dz, 0)
            pltpu.sync_copy(ast_hbm.at[head], as_v)
            pltpu.sync_copy(adt_hbm.at[head], ad_v)

            def p1_block(b, _):
                stage(b)

                def p1_chunk(j, _):
                    for v in range(CH // L):
                        sl = pl.ds(v * L, L)
                        d16 = dst_v[j, sl]
                        plsc.addupdate_scatter(
                            den_v, [d16], edge_e(src_v[j, sl], d16))
                    return 0
                lax.fori_loop(0, BCH, p1_chunk, 0)
                return 0
            lax.fori_loop(0, NBLK, p1_block, 0)

            pltpu.sync_copy(den_v, dsp.at[s])
            plsc.subcore_barrier()
            col = pl.ds(s * SL, SL)
            for quarter in range(4):
                pltpu.sync_copy(
                    dsp.at[pl.ds(quarter * (NS // 4), NS // 4), col], mrg_v)

                def dred(v2, _):
                    sl = pl.ds(v2 * L, L)
                    acc = (mrg_v[0, sl] if quarter == 0
                           else red_v[sl] + mrg_v[0, sl])
                    for r in range(1, NS // 4):
                        acc = acc + mrg_v[r, sl]
                    red_v[sl] = acc
                    return 0
                lax.fori_loop(0, SL // L, dred, 0)
            pltpu.sync_copy(red_v, mer.at[col])
            plsc.subcore_barrier()
            pltpu.sync_copy(mer, den_v)

            def drcp(i, _):
                sl = pl.ds(i * L, L)
                den_v[sl] = scale / (den_v[sl] + 1e-16)
                return 0
            lax.fori_loop(0, NPAD // L, drcp, 0)

            off = head * NPAD

            def coef_chunk(j, bb):
                for v in range(CH // L):
                    sl = pl.ds(v * L, L)
                    s16 = src_v[j, sl]
                    d16 = dst_v[j, sl]
                    rcp = plsc.load_gather(den_v, [d16])
                    cf_v[bb, sl] = edge_e(s16, d16) * rcp
                    adj_v[bb, sl] = s16 + off

            def fire_gather(bb):
                pltpu.async_copy(ht_hbm.at[adj_v.at[bb]], rows[bb], sems[bb])

            def wait_gather(bb):
                pltpu.make_async_copy(ht_hbm.at[pl.ds(0, CH), :],
                                      rows[bb], sems[bb]).wait()

            def scale_scatter(j, bb):
                rv = rows[bb]
                himask = jnp.full((L,), -65536, dtype=i32)

                def rscale(i2, _):
                    for u in range(2):
                        i = i2 * 2 + u
                        bc = plsc.load_gather(
                            cf_v.at[bb], [jnp.full((L,), i, dtype=i32)])
                        for g in range(HID // 32):
                            w = rv[i, pl.ds(g * L, L)]
                            lo = lax.bitcast_convert_type(
                                lax.shift_left(w, 16), f32)
                            hi = lax.bitcast_convert_type(w & himask, f32)
                            rowf_v[i, pl.ds(32 * g, L)] = lo * bc
                            rowf_v[i, pl.ds(32 * g + L, L)] = hi * bc
                    return 0
                lax.fori_loop(0, CH // 2, rscale, 0)
                pltpu.sync_copy(rowf_v, out_sp.at[dst_v.at[j]], add=True)

            def p2_block(b, _):
                stage(b)
                for bb in range(3):
                    coef_chunk(bb, bb)
                    fire_gather(bb)

                def p2_quad(j4, _):
                    for u in range(4):
                        j = 4 * j4 + u
                        bb = u
                        wait_gather(bb)
                        nb = (u + 3) % 4
                        if u == 0:
                            coef_chunk(j + 3, nb)
                            fire_gather(nb)
                        else:
                            @pl.when(j4 < BCH // 4 - 1)
                            def _():
                                coef_chunk(j + 3, nb)
                                fire_gather(nb)
                        scale_scatter(j, bb)
                    return 0
                lax.fori_loop(0, BCH // 4, p2_quad, 0)
                return 0
            lax.fori_loop(0, NBLK, p2_block, 0)

            if concat:
                plsc.subcore_barrier()
                pltpu.sync_copy(out_sp.at[pl.ds(s * SL, SL), :],
                                out_hbm.at[head, pl.ds(s * SL, SL), :])
                zero_out_slice()
            return 0

        lax.fori_loop(0, HPC, head_step, 0)

        if not concat:
            plsc.subcore_barrier()
            pltpu.sync_copy(out_sp.at[pl.ds(s * SL, SL), :],
                            out_hbm.at[c, pl.ds(s * SL, SL), :])

    return sck


_sc_layer1 = _make_sc_layer(concat=True)
_sc_layer2 = _make_sc_layer(concat=False)


def kernel(x, edge_index, W1, a_src1, a_dst1, b1, W2, a_src2, a_dst2, b2):
    xp = _pad_x(x)
    src2d, dst2d = _pad_edges(edge_index.reshape(2, E // CH, CH))
    z64 = jnp.zeros((CH, HID), f32)

    tb1, ast1, adt1 = _prep1(xp, W1, a_src1, a_dst1)
    out1 = _sc_layer1(src2d, dst2d, ast1, adt1,
                      tb1.reshape(HEADS * NPAD, HID // 2), z64)
    tb2, ast2, adt2 = _prep2(out1, b1.reshape(HEADS, HID),
                             W2.reshape(HEADS, HID, HEADS * C),
                             a_src2, a_dst2)
    part = _sc_layer2(src2d, dst2d, ast2, adt2,
                      tb2.reshape(HEADS * NPAD, C // 2), z64)
    return _final(part, b2.reshape(1, C))


# speedup vs baseline: 24.1572x; 1.1614x over previous
import functools

import jax
import jax.numpy as jnp
from jax import lax
from jax.experimental import pallas as pl
from jax.experimental.pallas import tpu as pltpu
from jax.experimental.pallas import tpu_sc as plsc

N = 10000
E = 320000
F_IN = 128
HID = 64
HEADS = 8
C = 64

NPAD = 10240
EPAD = 327680
NC, NS, L = 2, 16, 16
SL = NPAD // NS
EPT = EPAD // NS
CH = 128
SBLK = 2048
NBLK = EPT // SBLK
BCH = SBLK // CH
HPC = HEADS // NC
NEG_SLOPE = 0.2

f32 = jnp.float32
i32 = jnp.int32


def _pack_rows(blk):
    parts = []
    for g in range(blk.shape[1] // 32):
        a = blk[:, 32 * g:32 * g + 16].astype(jnp.bfloat16)
        b = blk[:, 32 * g + 16:32 * g + 32].astype(jnp.bfloat16)
        ai = lax.bitcast_convert_type(a, jnp.uint16).astype(i32)
        bi = lax.bitcast_convert_type(b, jnp.uint16).astype(i32)
        parts.append(ai | (bi << 16))
    return jnp.concatenate(parts, axis=1)


def _prep1_body(x_ref, w_ref, asr_ref, adr_ref, tb_ref, ast_ref, adt_ref):
    h = jnp.dot(x_ref[...], w_ref[...], preferred_element_type=f32)
    for hh in range(HEADS):
        blk = h[:, hh * HID:(hh + 1) * HID]
        tb_ref[hh] = _pack_rows(blk)
        ast_ref[hh] = jnp.sum(blk * asr_ref[hh][None, :], axis=1)
        adt_ref[hh] = jnp.sum(blk * adr_ref[hh][None, :], axis=1)


def _prep1(xp, W1, a_src, a_dst):
    BR = 1024
    nb = NPAD // BR
    return pl.pallas_call(
        _prep1_body,
        grid=(nb,),
        in_specs=[
            pl.BlockSpec((BR, F_IN), lambda i: (i, 0)),
            pl.BlockSpec((F_IN, HEADS * HID), lambda i: (0, 0)),
            pl.BlockSpec((HEADS, HID), lambda i: (0, 0)),
            pl.BlockSpec((HEADS, HID), lambda i: (0, 0)),
        ],
        out_specs=[
            pl.BlockSpec((HEADS, BR, HID // 2), lambda i: (0, i, 0)),
            pl.BlockSpec((HEADS, BR), lambda i: (0, i)),
            pl.BlockSpec((HEADS, BR), lambda i: (0, i)),
        ],
        out_shape=[
            jax.ShapeDtypeStruct((HEADS, NPAD, HID // 2), i32),
            jax.ShapeDtypeStruct((HEADS, NPAD), f32),
            jax.ShapeDtypeStruct((HEADS, NPAD), f32),
        ],
    )(xp, W1, a_src, a_dst)


def _prep2_body(o1_ref, b1_ref, w2_ref, asr_ref, adr_ref,
                ht_ref, ast_ref, adt_ref):
    acc = jnp.zeros((o1_ref.shape[1], HEADS * C), f32)
    for hh in range(HEADS):
        v = o1_ref[hh] + b1_ref[hh][None, :]
        v = jnp.where(v > 0, v, jnp.exp(v) - 1.0)
        acc = acc + jnp.dot(v, w2_ref[hh], preferred_element_type=f32)
    for hh in range(HEADS):
        blk = acc[:, hh * C:(hh + 1) * C]
        ht_ref[hh] = _pack_rows(blk)
        ast_ref[hh] = jnp.sum(blk * asr_ref[hh][None, :], axis=1)
        adt_ref[hh] = jnp.sum(blk * adr_ref[hh][None, :], axis=1)


def _prep2(out1, b1r, W2r, a_src, a_dst):
    BR = 1024
    nb = NPAD // BR
    return pl.pallas_call(
        _prep2_body,
        grid=(nb,),
        in_specs=[
            pl.BlockSpec((HEADS, BR, HID), lambda i: (0, i, 0)),
            pl.BlockSpec((HEADS, HID), lambda i: (0, 0)),
            pl.BlockSpec((HEADS, HID, HEADS * C), lambda i: (0, 0, 0)),
            pl.BlockSpec((HEADS, C), lambda i: (0, 0)),
            pl.BlockSpec((HEADS, C), lambda i: (0, 0)),
        ],
        out_specs=[
            pl.BlockSpec((HEADS, BR, C // 2), lambda i: (0, i, 0)),
            pl.BlockSpec((HEADS, BR), lambda i: (0, i)),
            pl.BlockSpec((HEADS, BR), lambda i: (0, i)),
        ],
        out_shape=[
            jax.ShapeDtypeStruct((HEADS, NPAD, C // 2), i32),
            jax.ShapeDtypeStruct((HEADS, NPAD), f32),
            jax.ShapeDtypeStruct((HEADS, NPAD), f32),
        ],
    )(out1, b1r, W2r, a_src, a_dst)


def _final_body(p_ref, b2_ref, o_ref):
    o_ref[...] = p_ref[0] + p_ref[1] + b2_ref[...]


def _final(part, b2r):
    BR = 1000
    nb = N // BR
    return pl.pallas_call(
        _final_body,
        grid=(nb,),
        in_specs=[
            pl.BlockSpec((2, BR, C), lambda i: (0, i, 0)),
            pl.BlockSpec((1, C), lambda i: (0, 0)),
        ],
        out_specs=pl.BlockSpec((BR, C), lambda i: (i, 0)),
        out_shape=jax.ShapeDtypeStruct((N, C), f32),
    )(part, b2r)


def _pad_x_body(x_ref, o_ref):
    o_ref[pl.ds(0, N), :] = x_ref[...]
    o_ref[pl.ds(N, NPAD - N), :] = jnp.zeros((NPAD - N, F_IN), f32)


def _pad_x(x):
    return pl.pallas_call(
        _pad_x_body,
        out_shape=jax.ShapeDtypeStruct((NPAD, F_IN), f32),
    )(x)


def _pad_edges_body(e_ref, s_ref, d_ref):
    fill = jnp.full((EPAD // CH - E // CH, CH), N, dtype=i32)
    s_ref[pl.ds(0, E // CH), :] = e_ref[0]
    s_ref[pl.ds(E // CH, EPAD // CH - E // CH), :] = fill
    d_ref[pl.ds(0, E // CH), :] = e_ref[1]
    d_ref[pl.ds(E // CH, EPAD // CH - E // CH), :] = fill


def _pad_edges(ei3):
    return pl.pallas_call(
        _pad_edges_body,
        out_shape=[
            jax.ShapeDtypeStruct((EPAD // CH, CH), i32),
            jax.ShapeDtypeStruct((EPAD // CH, CH), i32),
        ],
    )(ei3)


def _make_sc_layer(concat):
    scale = 1.0 if concat else 1.0 / HEADS
    out_shape = (jax.ShapeDtypeStruct((HEADS, NPAD, HID), f32) if concat
                 else jax.ShapeDtypeStruct((NC, NPAD, C), f32))
    mesh = plsc.VectorSubcoreMesh(core_axis_name="c", subcore_axis_name="s",
                                  num_cores=NC, num_subcores=NS)

    @functools.partial(
        pl.kernel, mesh=mesh, out_type=out_shape,
        compiler_params=pltpu.CompilerParams(needs_layout_passes=False,
                                             use_tc_tiling_on_sc=False),
        scratch_types=[
            pltpu.VMEM((BCH, CH), i32),
            pltpu.VMEM((BCH, CH), i32),
            pltpu.VMEM((4, CH), i32),
            pltpu.VMEM((4, CH), f32),
            pltpu.VMEM((NPAD,), f32),
            pltpu.VMEM((NPAD,), f32),
            pltpu.VMEM((NPAD,), f32),
            pltpu.VMEM((NS // 4, SL), f32),
            pltpu.VMEM((SL,), f32),
            pltpu.VMEM((CH, HID // 2), i32),
            pltpu.VMEM((CH, HID // 2), i32),
            pltpu.VMEM((CH, HID // 2), i32),
            pltpu.VMEM((CH, HID // 2), i32),
            pltpu.VMEM((CH, HID), f32),
            pltpu.VMEM((CH, HID), f32),
            pltpu.VMEM_SHARED((NPAD, HID), f32),
            pltpu.VMEM_SHARED((NS, NPAD), f32),
            pltpu.VMEM_SHARED((NPAD,), f32),
            pltpu.SemaphoreType.DMA,
            pltpu.SemaphoreType.DMA,
            pltpu.SemaphoreType.DMA,
            pltpu.SemaphoreType.DMA,
            pltpu.SemaphoreType.DMA,
            pltpu.SemaphoreType.DMA,
        ],
    )
    def sck(src_hbm, dst_hbm, ast_hbm, adt_hbm, ht_hbm, z_hbm, out_hbm,
            src_v, dst_v, adj_v, cf_v, as_v, ad_v, den_v, mrg_v, red_v,
            rows0_v, rows1_v, rows2_v, rows3_v, rowf0_v, rowf1_v,
            out_sp, dsp, mer, sem0, sem1, sem2, sem3, ssem0, ssem1):
        rows = (rows0_v, rows1_v, rows2_v, rows3_v)
        sems = (sem0, sem1, sem2, sem3)
        rowf = (rowf0_v, rowf1_v)
        ssems = (ssem0, ssem1)
        c = lax.axis_index("c")
        s = lax.axis_index("s")
        zero16 = jnp.zeros((L,), f32)

        def zero_out_slice():
            for k in range(SL // CH):
                pltpu.sync_copy(z_hbm,
                                out_sp.at[pl.ds(s * SL + k * CH, CH), :])
        zero_out_slice()

        def stage(b):
            row0 = s * (EPT // CH) + b * BCH
            pltpu.sync_copy(src_hbm.at[pl.ds(row0, BCH), :], src_v)
            pltpu.sync_copy(dst_hbm.at[pl.ds(row0, BCH), :], dst_v)

        def edge_e(s16, d16):
            a = plsc.load_gather(as_v, [s16]) + plsc.load_gather(ad_v, [d16])
            a = jnp.where(a > 0, a, NEG_SLOPE * a)
            return jnp.exp(a)

        def head_step(hh, _):
            head = c * HPC + hh

            def dz(i, _):
                den_v[pl.ds(i * L, L)] = zero16
                return 0
            lax.fori_loop(0, NPAD // L, dz, 0)
            pltpu.sync_copy(ast_hbm.at[head], as_v)
            pltpu.sync_copy(adt_hbm.at[head], ad_v)

            def p1_block(b, _):
                stage(b)

                def p1_chunk(j, _):
                    for v in range(CH // L):
                        sl = pl.ds(v * L, L)
                        d16 = dst_v[j, sl]
                        plsc.addupdate_scatter(
                            den_v, [d16], edge_e(src_v[j, sl], d16))
                    return 0
                lax.fori_loop(0, BCH, p1_chunk, 0)
                return 0
            lax.fori_loop(0, NBLK, p1_block, 0)

            pltpu.sync_copy(den_v, dsp.at[s])
            plsc.subcore_barrier()
            col = pl.ds(s * SL, SL)
            for quarter in range(4):
                pltpu.sync_copy(
                    dsp.at[pl.ds(quarter * (NS // 4), NS // 4), col], mrg_v)

                def dred(v2, _):
                    sl = pl.ds(v2 * L, L)
                    acc = (mrg_v[0, sl] if quarter == 0
                           else red_v[sl] + mrg_v[0, sl])
                    for r in range(1, NS // 4):
                        acc = acc + mrg_v[r, sl]
                    red_v[sl] = acc
                    return 0
                lax.fori_loop(0, SL // L, dred, 0)
            pltpu.sync_copy(red_v, mer.at[col])
            plsc.subcore_barrier()
            pltpu.sync_copy(mer, den_v)

            def drcp(i, _):
                sl = pl.ds(i * L, L)
                den_v[sl] = scale / (den_v[sl] + 1e-16)
                return 0
            lax.fori_loop(0, NPAD // L, drcp, 0)

            off = head * NPAD

            def coef_chunk(j, bb):
                for v in range(CH // L):
                    sl = pl.ds(v * L, L)
                    s16 = src_v[j, sl]
                    d16 = dst_v[j, sl]
                    rcp = plsc.load_gather(den_v, [d16])
                    cf_v[bb, sl] = edge_e(s16, d16) * rcp
                    adj_v[bb, sl] = s16 + off

            def fire_gather(bb):
                pltpu.async_copy(ht_hbm.at[adj_v.at[bb]], rows[bb], sems[bb])

            def wait_gather(bb):
                pltpu.make_async_copy(ht_hbm.at[pl.ds(0, CH), :],
                                      rows[bb], sems[bb]).wait()

            def wait_scatter(sb):
                pltpu.make_async_copy(z_hbm, rowf[sb], ssems[sb]).wait()

            def scale_scatter(j, bb, sb):
                rv = rows[bb]
                rf = rowf[sb]
                himask = jnp.full((L,), -65536, dtype=i32)

                def rscale(v, _):
                    cfv = cf_v[bb, pl.ds(v * L, L)]
                    for u in range(L):
                        i = v * L + u
                        bc = lax.gather(
                            cfv, jnp.full((L, 1), u, dtype=i32),
                            dimension_numbers=lax.GatherDimensionNumbers(
                                offset_dims=(), collapsed_slice_dims=(0,),
                                start_index_map=(0,)),
                            slice_sizes=(1,),
                            mode=lax.GatherScatterMode.PROMISE_IN_BOUNDS)
                        for g in range(HID // 32):
                            w = rv[i, pl.ds(g * L, L)]
                            lo = lax.bitcast_convert_type(
                                lax.shift_left(w, 16), f32)
                            hi = lax.bitcast_convert_type(w & himask, f32)
                            rf[i, pl.ds(32 * g, L)] = lo * bc
                            rf[i, pl.ds(32 * g + L, L)] = hi * bc
                    return 0
                lax.fori_loop(0, CH // L, rscale, 0)
                pltpu.async_copy(rowf[sb], out_sp.at[dst_v.at[j]], ssems[sb],
                                 add=True)

            def p2_block(b, _):
                stage(b)
                for bb in range(3):
                    coef_chunk(bb, bb)
                    fire_gather(bb)

                def p2_quad(j4, _):
                    for u in range(4):
                        j = 4 * j4 + u
                        bb = u
                        sb = u % 2
                        wait_gather(bb)
                        nb = (u + 3) % 4
                        if u == 0:
                            coef_chunk(j + 3, nb)
                            fire_gather(nb)
                        else:
                            @pl.when(j4 < BCH // 4 - 1)
                            def _():
                                coef_chunk(j + 3, nb)
                                fire_gather(nb)
                        if u < 2:
                            @pl.when(j4 > 0)
                            def _():
                                wait_scatter(sb)
                        else:
                            wait_scatter(sb)
                        scale_scatter(j, bb, sb)
                    return 0
                lax.fori_loop(0, BCH // 4, p2_quad, 0)
                wait_scatter(0)
                wait_scatter(1)
                return 0
            lax.fori_loop(0, NBLK, p2_block, 0)

            if concat:
                plsc.subcore_barrier()
                pltpu.sync_copy(out_sp.at[pl.ds(s * SL, SL), :],
                                out_hbm.at[head, pl.ds(s * SL, SL), :])
                zero_out_slice()
            return 0

        lax.fori_loop(0, HPC, head_step, 0)

        if not concat:
            plsc.subcore_barrier()
            pltpu.sync_copy(out_sp.at[pl.ds(s * SL, SL), :],
                            out_hbm.at[c, pl.ds(s * SL, SL), :])

    return sck


_sc_layer1 = _make_sc_layer(concat=True)
_sc_layer2 = _make_sc_layer(concat=False)


def kernel(x, edge_index, W1, a_src1, a_dst1, b1, W2, a_src2, a_dst2, b2):
    xp = _pad_x(x)
    src2d, dst2d = _pad_edges(edge_index.reshape(2, E // CH, CH))
    z64 = jnp.zeros((CH, HID), f32)

    tb1, ast1, adt1 = _prep1(xp, W1, a_src1, a_dst1)
    out1 = _sc_layer1(src2d, dst2d, ast1, adt1,
                      tb1.reshape(HEADS * NPAD, HID // 2), z64)
    tb2, ast2, adt2 = _prep2(out1, b1.reshape(HEADS, HID),
                             W2.reshape(HEADS, HID, HEADS * C),
                             a_src2, a_dst2)
    part = _sc_layer2(src2d, dst2d, ast2, adt2,
                      tb2.reshape(HEADS * NPAD, C // 2), z64)
    return _final(part, b2.reshape(1, C))
